# baseline (speedup 1.0000x reference)
import functools

import jax
import jax.numpy as jnp
from jax import lax
from jax.experimental import pallas as pl
from jax.experimental.pallas import tpu as pltpu
from jax.experimental.pallas import tpu_sc as plsc

N = 50000
D = 64
H = 32
NUM_LAYERS = 3
ALPHA = 1.0 / (NUM_LAYERS + 1)

NC = 2
NS = 16

NP = 50176
NT = NP // NS
CN = 32
NQ = NT // CN
GQ = 7
REM = N % CN
BQ = (N - (NS - 1) * NT) // CN
S = 4

E = 800000
EPT = 50176
E_PAD = EPT * NS
BLK = 128
G = 8
CB = EPT // BLK
NJ = CB // G
RB = E_PAD // BLK


def _rsqrt16(d):
    y = jnp.full((16,), 2.0 ** -11, jnp.float32)
    for k in range(9, -1, -1):
        y = jnp.where(d < 4.0 ** (k + 1), jnp.float32(2.0 ** -(k + 1)), y)
    for _ in range(6):
        y = y * (1.5 - 0.5 * d * y * y)
    return jnp.where(d > 0.5, y, 0.0)


def _propagate_body(emb, row2d, col2d, o, y0, y1,
                    xb, ob, dinvv, idx_r, idx_c, rows, ones_v,
                    acc_sh, deg_sh, gsem, ssem, osem, ysem):
    c = lax.axis_index("c")
    s = lax.axis_index("s")
    z16 = jnp.zeros((16,), jnp.float32)
    one16 = jnp.ones((16,), jnp.float32)

    obz = ob.at[0]

    def _zero_ob(i, _):
        obz[i, 0:16] = z16
        obz[i, 16:32] = z16
        return 0

    def _fill_ones(k, _):
        ones_v[pl.ds(k * 16, 16)] = one16
        return 0
    lax.fori_loop(0, BLK // 16, _fill_ones, 0)

    def _zero_dinvv(k, _):
        dinvv[pl.ds(k * 16, 16)] = z16
        return 0
    lax.fori_loop(0, NT // 16, _zero_dinvv, 0)
    pltpu.sync_copy(dinvv, deg_sh.at[pl.ds(s * NT, NT)])
    plsc.subcore_barrier()

    def _deg_chunk(j, _):
        cblk = s * CB + j * G
        pltpu.sync_copy(col2d.at[pl.ds(cblk, G)], idx_c)
        descs = [pltpu.async_copy(ones_v, deg_sh.at[idx_c.at[t]], ssem,
                                  add=True)
                 for t in range(G)]
        for dsc in descs:
            dsc.wait()
        return 0
    lax.fori_loop(0, NJ, _deg_chunk, 0)
    plsc.subcore_barrier()

    pltpu.sync_copy(deg_sh.at[pl.ds(s * NT, NT)], dinvv)

    def _dinv(k, _):
        dinvv[pl.ds(k * 16, 16)] = _rsqrt16(dinvv[pl.ds(k * 16, 16)])
        return 0
    lax.fori_loop(0, NT // 16, _dinv, 0)

    def _p0_loop(col0, y_ref):
        def _grp(jg, _):
            wo = [None] * GQ
            wy = [None] * GQ
            fulls = [None] * GQ
            for qq in range(GQ):
                q = jg * GQ + qq
                g = s * NT + q * CN
                full = g + CN <= N
                b = qq % 2
                if qq >= 2:
                    @pl.when(fulls[qq - 2])
                    def _(w=wo[qq - 2]):
                        w.wait()
                    wy[qq - 2].wait()
                xbb = xb.at[b]
                obb = ob.at[b]
                rd = pltpu.make_async_copy(
                    emb.at[pl.ds(g, CN), pl.ds(col0, H)], xbb, gsem)

                @pl.when(full)
                def _(r=rd):
                    r.start()
                    r.wait()
                wod = pltpu.make_async_copy(
                    xbb, o.at[pl.ds(g, CN), pl.ds(col0, H)], osem)

                @pl.when(full)
                def _(w=wod):
                    w.start()
                wo[qq] = wod
                fulls[qq] = full

                def _n(i, _, xbb=xbb, obb=obb, q=q):
                    li = q * CN + i
                    dv = plsc.load_gather(
                        dinvv, [jnp.full((16,), li, jnp.int32)])
                    obb[i, 0:16] = xbb[i, 0:16] * dv
                    obb[i, 16:32] = xbb[i, 16:32] * dv
                    return 0
                lax.fori_loop(0, CN, _n, 0)
                wy[qq] = pltpu.async_copy(obb, y_ref.at[pl.ds(g, CN)], ysem)
            for qq in range(GQ - 2, GQ):
                @pl.when(fulls[qq])
                def _(w=wo[qq]):
                    w.wait()
                wy[qq].wait()
            return 0
        lax.fori_loop(0, NQ // GQ, _grp, 0)

        @pl.when(s == NS - 1)
        def _():
            gb = (NS - 1) * NT + BQ * CN
            xbb = xb.at[0]
            pltpu.sync_copy(emb.at[pl.ds(gb, REM), pl.ds(col0, H)],
                            xbb.at[pl.ds(0, REM)])
            pltpu.sync_copy(xbb.at[pl.ds(0, REM)],
                            o.at[pl.ds(gb, REM), pl.ds(col0, H)])

            def _n(i, _):
                li = BQ * CN + i
                dv = plsc.load_gather(dinvv,
                                      [jnp.full((16,), li, jnp.int32)])
                xbb[i, 0:16] = xbb[i, 0:16] * dv
                xbb[i, 16:32] = xbb[i, 16:32] * dv
                return 0
            lax.fori_loop(0, REM, _n, 0)
            pltpu.sync_copy(xbb.at[pl.ds(0, REM)],
                            y_ref.at[pl.ds(gb, REM)])

    @pl.when(c == 0)
    def _():
        _p0_loop(0, y0)

    @pl.when(c == 1)
    def _():
        _p0_loop(H, y1)

    def _edge_loop(y_ref):
        def _edge_chunk(j, _):
            cblk = s * CB + j * G
            pltpu.sync_copy(row2d.at[pl.ds(cblk, G)], idx_r)
            pltpu.sync_copy(col2d.at[pl.ds(cblk, G)], idx_c)
            gd = [None] * G
            sd = [None] * G
            for t in range(G):
                if t >= S:
                    sd[t - S].wait()
                gd[t] = pltpu.async_copy(y_ref.at[idx_r.at[t]],
                                         rows.at[t % S], gsem)
                u = t - (S - 1)
                if u >= 0:
                    gd[u].wait()
                    sd[u] = pltpu.async_copy(rows.at[u % S],
                                             acc_sh.at[idx_c.at[u]],
                                             ssem, add=True)
            for u in range(G - (S - 1), G):
                gd[u].wait()
                sd[u] = pltpu.async_copy(rows.at[u % S],
                                         acc_sh.at[idx_c.at[u]],
                                         ssem, add=True)
            for u in range(G - S, G):
                if u >= 0:
                    sd[u].wait()
            return 0
        lax.fori_loop(0, NJ, _edge_chunk, 0)

    def _node_loop(col0, y_ref, last):
        def _grp(jg, _):
            wo = [None] * GQ
            wy = [None] * GQ
            fulls = [None] * GQ
            for qq in range(GQ):
                q = jg * GQ + qq
                g = s * NT + q * CN
                full = g + CN <= N
                b = qq % 2
                if qq >= 2:
                    @pl.when(fulls[qq - 2])
                    def _(w=wo[qq - 2]):
                        w.wait()
                    if not last:
                        wy[qq - 2].wait()
                xbb = xb.at[b]
                obb = ob.at[b]
                pltpu.async_copy(acc_sh.at[pl.ds(g, CN)], xbb, gsem).wait()
                ro = pltpu.make_async_copy(
                    o.at[pl.ds(g, CN), pl.ds(col0, H)], obb, gsem)

                @pl.when(full)
                def _(r=ro):
                    r.start()
                    r.wait()

                def _n(i, _, xbb=xbb, obb=obb, q=q):
                    li = q * CN + i
                    dv = plsc.load_gather(
                        dinvv, [jnp.full((16,), li, jnp.int32)])
                    x0 = xbb[i, 0:16] * dv
                    x1 = xbb[i, 16:32] * dv
                    o0_ = obb[i, 0:16] + x0
                    o1_ = obb[i, 16:32] + x1
                    if last:
                        o0_ = o0_ * ALPHA
                        o1_ = o1_ * ALPHA
                    else:
                        xbb[i, 0:16] = x0 * dv
                        xbb[i, 16:32] = x1 * dv
                    obb[i, 0:16] = o0_
                    obb[i, 16:32] = o1_
                    return 0
                lax.fori_loop(0, CN, _n, 0)
                wod = pltpu.make_async_copy(
                    obb, o.at[pl.ds(g, CN), pl.ds(col0, H)], osem)

                @pl.when(full)
                def _(w=wod):
                    w.start()
                wo[qq] = wod
                fulls[qq] = full
                if not last:
                    wy[qq] = pltpu.async_copy(xbb, y_ref.at[pl.ds(g, CN)],
                                              ysem)
            for qq in range(GQ - 2, GQ):
                @pl.when(fulls[qq])
                def _(w=wo[qq]):
                    w.wait()
                if not last:
                    wy[qq].wait()
            return 0
        lax.fori_loop(0, NQ // GQ, _grp, 0)

        @pl.when(s == NS - 1)
        def _():
            gb = (NS - 1) * NT + BQ * CN
            xbb = xb.at[0]
            obb = ob.at[0]
            pltpu.sync_copy(acc_sh.at[pl.ds(gb, REM)],
                            xbb.at[pl.ds(0, REM)])
            pltpu.sync_copy(o.at[pl.ds(gb, REM), pl.ds(col0, H)],
                            obb.at[pl.ds(0, REM)])

            def _n(i, _):
                li = BQ * CN + i
                dv = plsc.load_gather(dinvv,
                                      [jnp.full((16,), li, jnp.int32)])
                o0_ = obb[i, 0:16] + xbb[i, 0:16] * dv
                o1_ = obb[i, 16:32] + xbb[i, 16:32] * dv
                if last:
                    o0_ = o0_ * ALPHA
                    o1_ = o1_ * ALPHA
                obb[i, 0:16] = o0_
                obb[i, 16:32] = o1_
                return 0
            lax.fori_loop(0, REM, _n, 0)
            pltpu.sync_copy(obb.at[pl.ds(0, REM)],
                            o.at[pl.ds(gb, REM), pl.ds(col0, H)])

    for l in range(NUM_LAYERS):
        last = l == NUM_LAYERS - 1

        lax.fori_loop(0, CN, _zero_ob, 0)

        def _zero_acc(q, _):
            pltpu.sync_copy(ob.at[0], acc_sh.at[pl.ds(s * NT + q * CN, CN)])
            return 0
        lax.fori_loop(0, NQ, _zero_acc, 0)
        plsc.subcore_barrier()

        @pl.when(c == 0)
        def _():
            _edge_loop(y0)

        @pl.when(c == 1)
        def _():
            _edge_loop(y1)
        plsc.subcore_barrier()

        @pl.when(c == 0)
        def _():
            _node_loop(0, y0, last)

        @pl.when(c == 1)
        def _():
            _node_loop(H, y1, last)


_propagate = functools.partial(
    pl.kernel,
    out_type=[
        jax.ShapeDtypeStruct((N, D), jnp.float32),
        jax.ShapeDtypeStruct((NP, H), jnp.float32),
        jax.ShapeDtypeStruct((NP, H), jnp.float32),
    ],
    mesh=plsc.VectorSubcoreMesh(core_axis_name="c", subcore_axis_name="s"),
    compiler_params=pltpu.CompilerParams(
        needs_layout_passes=False, use_tc_tiling_on_sc=False),
    scratch_types=[
        pltpu.VMEM((2, CN, H), jnp.float32),
        pltpu.VMEM((2, CN, H), jnp.float32),
        pltpu.VMEM((NT,), jnp.float32),
        pltpu.VMEM((G, BLK), jnp.int32),
        pltpu.VMEM((G, BLK), jnp.int32),
        pltpu.VMEM((S, BLK, H), jnp.float32),
        pltpu.VMEM((BLK,), jnp.float32),
        pltpu.VMEM_SHARED((NP, H), jnp.float32),
        pltpu.VMEM_SHARED((NP,), jnp.float32),
        pltpu.SemaphoreType.DMA,
        pltpu.SemaphoreType.DMA,
        pltpu.SemaphoreType.DMA,
        pltpu.SemaphoreType.DMA,
    ],
)(_propagate_body)


def kernel(emb, edge_index):
    row = edge_index[0].astype(jnp.int32)
    col = edge_index[1].astype(jnp.int32)
    pad = E_PAD - E
    rp = jnp.concatenate([row, jnp.zeros((pad,), jnp.int32)])
    cp = jnp.concatenate([col, jnp.full((pad,), N, jnp.int32)])
    row2d = rp.reshape(RB, BLK)
    col2d = cp.reshape(RB, BLK)
    o, _, _ = _propagate(emb, row2d, col2d)
    return o


# speedup vs baseline: 1.1666x; 1.1666x over previous
import functools

import jax
import jax.numpy as jnp
from jax import lax
from jax.experimental import pallas as pl
from jax.experimental.pallas import tpu as pltpu
from jax.experimental.pallas import tpu_sc as plsc

N = 50000
D = 64
H = 32
NUM_LAYERS = 3
ALPHA = 1.0 / (NUM_LAYERS + 1)

NC = 2
NS = 16

NP = 50176
NT = NP // NS
CN = 64
NQ = NT // CN
REM = N % CN
S = 4

E = 800000
EPT = 50176
E_PAD = EPT * NS
BLK = 128
G = 14
CB = EPT // BLK
NJ = CB // G
RB = E_PAD // BLK


def _rsqrt16(d):
    y = jnp.full((16,), 2.0 ** -11, jnp.float32)
    for k in range(9, -1, -1):
        y = jnp.where(d < 4.0 ** (k + 1), jnp.float32(2.0 ** -(k + 1)), y)
    for _ in range(6):
        y = y * (1.5 - 0.5 * d * y * y)
    return jnp.where(d > 0.5, y, 0.0)


def _propagate_body(emb, row2d, col2d, o, y0, y1,
                    xb, ob, dinvv, idx_r, idx_c, rows, ones_v,
                    acc_sh, deg_sh, gsem, ssem):
    c = lax.axis_index("c")
    s = lax.axis_index("s")
    z16 = jnp.zeros((16,), jnp.float32)
    one16 = jnp.ones((16,), jnp.float32)

    def _zero_ob(i, _):
        ob[i, 0:16] = z16
        ob[i, 16:32] = z16
        return 0

    def _fill_ones(k, _):
        ones_v[pl.ds(k * 16, 16)] = one16
        return 0
    lax.fori_loop(0, BLK // 16, _fill_ones, 0)

    def _zero_dinvv(k, _):
        dinvv[pl.ds(k * 16, 16)] = z16
        return 0
    lax.fori_loop(0, NT // 16, _zero_dinvv, 0)
    pltpu.sync_copy(dinvv, deg_sh.at[pl.ds(s * NT, NT)])
    plsc.subcore_barrier()

    def _deg_chunk(j, _):
        cblk = s * CB + j * G
        pltpu.sync_copy(col2d.at[pl.ds(cblk, G)], idx_c)
        descs = [pltpu.async_copy(ones_v, deg_sh.at[idx_c.at[t]], ssem,
                                  add=True)
                 for t in range(G)]
        for dsc in descs:
            dsc.wait()
        return 0
    lax.fori_loop(0, NJ, _deg_chunk, 0)
    plsc.subcore_barrier()

    pltpu.sync_copy(deg_sh.at[pl.ds(s * NT, NT)], dinvv)

    def _dinv(k, _):
        dinvv[pl.ds(k * 16, 16)] = _rsqrt16(dinvv[pl.ds(k * 16, 16)])
        return 0
    lax.fori_loop(0, NT // 16, _dinv, 0)

    def _p0_loop(col0, y_ref):
        def _p0(q, _):
            g = s * NT + q * CN

            @pl.when(g + CN <= N)
            def _():
                pltpu.sync_copy(emb.at[pl.ds(g, CN), pl.ds(col0, H)], xb)
                pltpu.sync_copy(xb, o.at[pl.ds(g, CN), pl.ds(col0, H)])

            @pl.when(jnp.logical_and(g + CN > N, g < N))
            def _():
                pltpu.sync_copy(emb.at[pl.ds(g, REM), pl.ds(col0, H)],
                                xb.at[pl.ds(0, REM)])
                pltpu.sync_copy(xb.at[pl.ds(0, REM)],
                                o.at[pl.ds(g, REM), pl.ds(col0, H)])

            def _n(i, _):
                li = q * CN + i
                dv = plsc.load_gather(dinvv,
                                      [jnp.full((16,), li, jnp.int32)])
                xb[i, 0:16] = xb[i, 0:16] * dv
                xb[i, 16:32] = xb[i, 16:32] * dv
                return 0
            lax.fori_loop(0, CN, _n, 0)
            pltpu.sync_copy(xb, y_ref.at[pl.ds(g, CN)])
            return 0
        lax.fori_loop(0, NQ, _p0, 0)

    @pl.when(c == 0)
    def _():
        _p0_loop(0, y0)

    @pl.when(c == 1)
    def _():
        _p0_loop(H, y1)

    def _edge_loop(y_ref):
        def _edge_chunk(j, _):
            cblk = s * CB + j * G
            pltpu.sync_copy(row2d.at[pl.ds(cblk, G)], idx_r)
            pltpu.sync_copy(col2d.at[pl.ds(cblk, G)], idx_c)
            gd = [None] * G
            sd = [None] * G
            for t in range(G):
                if t >= S:
                    sd[t - S].wait()
                gd[t] = pltpu.async_copy(y_ref.at[idx_r.at[t]],
                                         rows.at[t % S], gsem)
                u = t - (S - 1)
                if u >= 0:
                    gd[u].wait()
                    sd[u] = pltpu.async_copy(rows.at[u % S],
                                             acc_sh.at[idx_c.at[u]],
                                             ssem, add=True)
            for u in range(G - (S - 1), G):
                gd[u].wait()
                sd[u] = pltpu.async_copy(rows.at[u % S],
                                         acc_sh.at[idx_c.at[u]],
                                         ssem, add=True)
            for u in range(G - S, G):
                if u >= 0:
                    sd[u].wait()
            return 0
        lax.fori_loop(0, NJ, _edge_chunk, 0)

    def _node_loop(col0, y_ref, last):
        def _npass(q, _):
            g = s * NT + q * CN
            pltpu.sync_copy(acc_sh.at[pl.ds(g, CN)], xb)

            @pl.when(g + CN <= N)
            def _():
                pltpu.sync_copy(o.at[pl.ds(g, CN), pl.ds(col0, H)], ob)

            @pl.when(jnp.logical_and(g + CN > N, g < N))
            def _():
                pltpu.sync_copy(o.at[pl.ds(g, REM), pl.ds(col0, H)],
                                ob.at[pl.ds(0, REM)])

            def _n(i, _):
                li = q * CN + i
                dv = plsc.load_gather(dinvv,
                                      [jnp.full((16,), li, jnp.int32)])
                x0 = xb[i, 0:16] * dv
                x1 = xb[i, 16:32] * dv
                o0_ = ob[i, 0:16] + x0
                o1_ = ob[i, 16:32] + x1
                if last:
                    o0_ = o0_ * ALPHA
                    o1_ = o1_ * ALPHA
                else:
                    xb[i, 0:16] = x0 * dv
                    xb[i, 16:32] = x1 * dv
                ob[i, 0:16] = o0_
                ob[i, 16:32] = o1_
                return 0
            lax.fori_loop(0, CN, _n, 0)

            @pl.when(g + CN <= N)
            def _():
                pltpu.sync_copy(ob, o.at[pl.ds(g, CN), pl.ds(col0, H)])

            @pl.when(jnp.logical_and(g + CN > N, g < N))
            def _():
                pltpu.sync_copy(ob.at[pl.ds(0, REM)],
                                o.at[pl.ds(g, REM), pl.ds(col0, H)])

            if not last:
                pltpu.sync_copy(xb, y_ref.at[pl.ds(g, CN)])
            return 0
        lax.fori_loop(0, NQ, _npass, 0)

    for l in range(NUM_LAYERS):
        last = l == NUM_LAYERS - 1

        lax.fori_loop(0, CN, _zero_ob, 0)

        def _zero_acc(q, _):
            pltpu.sync_copy(ob, acc_sh.at[pl.ds(s * NT + q * CN, CN)])
            return 0
        lax.fori_loop(0, NQ, _zero_acc, 0)
        plsc.subcore_barrier()

        @pl.when(c == 0)
        def _():
            _edge_loop(y0)

        @pl.when(c == 1)
        def _():
            _edge_loop(y1)
        plsc.subcore_barrier()

        @pl.when(c == 0)
        def _():
            _node_loop(0, y0, last)

        @pl.when(c == 1)
        def _():
            _node_loop(H, y1, last)


_propagate = functools.partial(
    pl.kernel,
    out_type=[
        jax.ShapeDtypeStruct((N, D), jnp.float32),
        jax.ShapeDtypeStruct((NP, H), jnp.float32),
        jax.ShapeDtypeStruct((NP, H), jnp.float32),
    ],
    mesh=plsc.VectorSubcoreMesh(core_axis_name="c", subcore_axis_name="s"),
    compiler_params=pltpu.CompilerParams(
        needs_layout_passes=False, use_tc_tiling_on_sc=False),
    scratch_types=[
        pltpu.VMEM((CN, H), jnp.float32),
        pltpu.VMEM((CN, H), jnp.float32),
        pltpu.VMEM((NT,), jnp.float32),
        pltpu.VMEM((G, BLK), jnp.int32),
        pltpu.VMEM((G, BLK), jnp.int32),
        pltpu.VMEM((S, BLK, H), jnp.float32),
        pltpu.VMEM((BLK,), jnp.float32),
        pltpu.VMEM_SHARED((NP, H), jnp.float32),
        pltpu.VMEM_SHARED((NP,), jnp.float32),
        pltpu.SemaphoreType.DMA,
        pltpu.SemaphoreType.DMA,
    ],
)(_propagate_body)


def kernel(emb, edge_index):
    row = edge_index[0].astype(jnp.int32)
    col = edge_index[1].astype(jnp.int32)
    pad = E_PAD - E
    rp = jnp.concatenate([row, jnp.zeros((pad,), jnp.int32)])
    cp = jnp.concatenate([col, jnp.full((pad,), N, jnp.int32)])
    row2d = rp.reshape(RB, BLK)
    col2d = cp.reshape(RB, BLK)
    o, _, _ = _propagate(emb, row2d, col2d)
    return o


# speedup vs baseline: 1.2092x; 1.0365x over previous
import functools

import jax
import jax.numpy as jnp
from jax import lax
from jax.experimental import pallas as pl
from jax.experimental.pallas import tpu as pltpu
from jax.experimental.pallas import tpu_sc as plsc

N = 50000
D = 64
H = 32
NUM_LAYERS = 3
ALPHA = 1.0 / (NUM_LAYERS + 1)

NC = 2
NS = 16

NP = 50176
NT = NP // NS
CN = 64
NQ = NT // CN
REM = N % CN
S = 4

E = 800000
EPT = 50176
E_PAD = EPT * NS
BLK = 128
G = 14
CB = EPT // BLK
NJ = CB // G
RB = E_PAD // BLK


def _rsqrt16(d):
    y = jnp.full((16,), 2.0 ** -11, jnp.float32)
    for k in range(9, -1, -1):
        y = jnp.where(d < 4.0 ** (k + 1), jnp.float32(2.0 ** -(k + 1)), y)
    for _ in range(6):
        y = y * (1.5 - 0.5 * d * y * y)
    return jnp.where(d > 0.5, y, 0.0)


def _propagate_body(emb, row2d, col2d, o, y0, y1,
                    xb, ob, dinvv, idx_r, idx_c, rows, ones_v,
                    acc_sh, deg_sh, gsem, ssem):
    c = lax.axis_index("c")
    s = lax.axis_index("s")
    z16 = jnp.zeros((16,), jnp.float32)
    one16 = jnp.ones((16,), jnp.float32)

    def _zero_ob(i, _):
        ob[i, 0:16] = z16
        ob[i, 16:32] = z16
        return 0

    def _fill_ones(k, _):
        ones_v[pl.ds(k * 16, 16)] = one16
        return 0
    lax.fori_loop(0, BLK // 16, _fill_ones, 0)

    def _zero_dinvv(k, _):
        dinvv[pl.ds(k * 16, 16)] = z16
        return 0
    lax.fori_loop(0, NT // 16, _zero_dinvv, 0)
    pltpu.sync_copy(dinvv, deg_sh.at[pl.ds(s * NT, NT)])
    plsc.subcore_barrier()

    def _deg_chunk(j, _):
        cblk = s * CB + j * G
        pltpu.sync_copy(col2d.at[pl.ds(cblk, G)], idx_c)
        descs = [pltpu.async_copy(ones_v, deg_sh.at[idx_c.at[t]], ssem,
                                  add=True)
                 for t in range(G)]
        for dsc in descs:
            dsc.wait()
        return 0
    lax.fori_loop(0, NJ, _deg_chunk, 0)
    plsc.subcore_barrier()

    pltpu.sync_copy(deg_sh.at[pl.ds(s * NT, NT)], dinvv)

    def _dinv(k, _):
        dinvv[pl.ds(k * 16, 16)] = _rsqrt16(dinvv[pl.ds(k * 16, 16)])
        return 0
    lax.fori_loop(0, NT // 16, _dinv, 0)

    def _p0_loop(col0, y_ref):
        def _p0(q, _):
            g = s * NT + q * CN

            @pl.when(g + CN <= N)
            def _():
                pltpu.sync_copy(emb.at[pl.ds(g, CN), pl.ds(col0, H)], xb)
                pltpu.sync_copy(xb, o.at[pl.ds(g, CN), pl.ds(col0, H)])

            @pl.when(jnp.logical_and(g + CN > N, g < N))
            def _():
                pltpu.sync_copy(emb.at[pl.ds(g, REM), pl.ds(col0, H)],
                                xb.at[pl.ds(0, REM)])
                pltpu.sync_copy(xb.at[pl.ds(0, REM)],
                                o.at[pl.ds(g, REM), pl.ds(col0, H)])

            def _n(i, _):
                li = q * CN + i
                dv = plsc.load_gather(dinvv,
                                      [jnp.full((16,), li, jnp.int32)])
                xb[i, 0:16] = xb[i, 0:16] * dv
                xb[i, 16:32] = xb[i, 16:32] * dv
                return 0
            lax.fori_loop(0, CN, _n, 0)
            pltpu.sync_copy(xb, y_ref.at[pl.ds(g, CN)])
            return 0
        lax.fori_loop(0, NQ, _p0, 0)

    @pl.when(c == 0)
    def _():
        _p0_loop(0, y0)

    @pl.when(c == 1)
    def _():
        _p0_loop(H, y1)

    def _edge_loop(y_ref):
        def _edge_chunk(j, _):
            cblk = s * CB + j * G
            ir = pltpu.async_copy(row2d.at[pl.ds(cblk, G)], idx_r, gsem)
            ic = pltpu.async_copy(col2d.at[pl.ds(cblk, G)], idx_c, gsem)
            ir.wait()
            ic.wait()
            gd = [None] * G
            sd = [None] * G
            for t in range(G):
                if t >= S:
                    sd[t - S].wait()
                gd[t] = pltpu.async_copy(y_ref.at[idx_r.at[t]],
                                         rows.at[t % S], gsem)
                u = t - (S - 1)
                if u >= 0:
                    gd[u].wait()
                    sd[u] = pltpu.async_copy(rows.at[u % S],
                                             acc_sh.at[idx_c.at[u]],
                                             ssem, add=True)
            for u in range(G - (S - 1), G):
                gd[u].wait()
                sd[u] = pltpu.async_copy(rows.at[u % S],
                                         acc_sh.at[idx_c.at[u]],
                                         ssem, add=True)
            for u in range(G - S, G):
                if u >= 0:
                    sd[u].wait()
            return 0
        lax.fori_loop(0, NJ, _edge_chunk, 0)

    def _node_loop(col0, y_ref, last):
        def _npass(q, _):
            g = s * NT + q * CN
            pltpu.sync_copy(acc_sh.at[pl.ds(g, CN)], xb)

            @pl.when(g + CN <= N)
            def _():
                pltpu.sync_copy(o.at[pl.ds(g, CN), pl.ds(col0, H)], ob)

            @pl.when(jnp.logical_and(g + CN > N, g < N))
            def _():
                pltpu.sync_copy(o.at[pl.ds(g, REM), pl.ds(col0, H)],
                                ob.at[pl.ds(0, REM)])

            def _n(i, _):
                li = q * CN + i
                dv = plsc.load_gather(dinvv,
                                      [jnp.full((16,), li, jnp.int32)])
                x0 = xb[i, 0:16] * dv
                x1 = xb[i, 16:32] * dv
                o0_ = ob[i, 0:16] + x0
                o1_ = ob[i, 16:32] + x1
                if last:
                    o0_ = o0_ * ALPHA
                    o1_ = o1_ * ALPHA
                else:
                    xb[i, 0:16] = x0 * dv
                    xb[i, 16:32] = x1 * dv
                ob[i, 0:16] = o0_
                ob[i, 16:32] = o1_
                return 0
            lax.fori_loop(0, CN, _n, 0)

            @pl.when(g + CN <= N)
            def _():
                pltpu.sync_copy(ob, o.at[pl.ds(g, CN), pl.ds(col0, H)])

            @pl.when(jnp.logical_and(g + CN > N, g < N))
            def _():
                pltpu.sync_copy(ob.at[pl.ds(0, REM)],
                                o.at[pl.ds(g, REM), pl.ds(col0, H)])

            if not last:
                pltpu.sync_copy(xb, y_ref.at[pl.ds(g, CN)])
            return 0
        lax.fori_loop(0, NQ, _npass, 0)

    for l in range(NUM_LAYERS):
        last = l == NUM_LAYERS - 1

        lax.fori_loop(0, CN, _zero_ob, 0)

        def _zero_acc(q, _):
            pltpu.sync_copy(ob, acc_sh.at[pl.ds(s * NT + q * CN, CN)])
            return 0
        lax.fori_loop(0, NQ, _zero_acc, 0)
        plsc.subcore_barrier()

        @pl.when(c == 0)
        def _():
            _edge_loop(y0)

        @pl.when(c == 1)
        def _():
            _edge_loop(y1)
        plsc.subcore_barrier()

        @pl.when(c == 0)
        def _():
            _node_loop(0, y0, last)

        @pl.when(c == 1)
        def _():
            _node_loop(H, y1, last)


_propagate = functools.partial(
    pl.kernel,
    out_type=[
        jax.ShapeDtypeStruct((N, D), jnp.float32),
        jax.ShapeDtypeStruct((NP, H), jnp.float32),
        jax.ShapeDtypeStruct((NP, H), jnp.float32),
    ],
    mesh=plsc.VectorSubcoreMesh(core_axis_name="c", subcore_axis_name="s"),
    compiler_params=pltpu.CompilerParams(
        needs_layout_passes=False, use_tc_tiling_on_sc=False),
    scratch_types=[
        pltpu.VMEM((CN, H), jnp.float32),
        pltpu.VMEM((CN, H), jnp.float32),
        pltpu.VMEM((NT,), jnp.float32),
        pltpu.VMEM((G, BLK), jnp.int32),
        pltpu.VMEM((G, BLK), jnp.int32),
        pltpu.VMEM((S, BLK, H), jnp.float32),
        pltpu.VMEM((BLK,), jnp.float32),
        pltpu.VMEM_SHARED((NP, H), jnp.float32),
        pltpu.VMEM_SHARED((NP,), jnp.float32),
        pltpu.SemaphoreType.DMA,
        pltpu.SemaphoreType.DMA,
    ],
)(_propagate_body)


def kernel(emb, edge_index):
    row = edge_index[0].astype(jnp.int32)
    col = edge_index[1].astype(jnp.int32)
    pad = E_PAD - E
    rp = jnp.concatenate([row, jnp.zeros((pad,), jnp.int32)])
    cp = jnp.concatenate([col, jnp.full((pad,), N, jnp.int32)])
    row2d = rp.reshape(RB, BLK)
    col2d = cp.reshape(RB, BLK)
    o, _, _ = _propagate(emb, row2d, col2d)
    return o
